# baseline (speedup 1.0000x reference)
import jax
import jax.numpy as jnp
from jax import lax
from jax.experimental import pallas as pl
from jax.experimental.pallas import tpu as pltpu
from jax.experimental.pallas import tpu_sc as plsc

NUM_COLLISIONS = 4
EMBED_DIM = 64
BATCH = 16384
HIST = 50

NC, NS, L = 2, 16, 16
NW = NC * NS
BAGS_PER_W = BATCH // NW
GB = 16
NG = BAGS_PER_W // GB
CH = HIST // 2
IDX_PER_C = GB * CH
N_SUB = 5
SUB = IDX_PER_C // N_SUB
DV = EMBED_DIM // L
PAIR = 2 * EMBED_DIM
NQP = 125000
RG = 8

_mesh = plsc.VectorSubcoreMesh(core_axis_name="c", subcore_axis_name="s")


@jax.jit
def _qr_bag(inp, weight_q, weight_r):
    inp_t = inp.T
    wq_pair = weight_q.reshape(NQP, PAIR)

    @pl.kernel(
        out_type=jax.ShapeDtypeStruct((EMBED_DIM, BATCH), jnp.float32),
        mesh=_mesh,
        compiler_params=pltpu.CompilerParams(
            needs_layout_passes=False, disable_bounds_checks=True),
        scratch_types=[
            pltpu.VMEM((HIST, RG * GB), jnp.int32),
            pltpu.VMEM((2, N_SUB, SUB), jnp.int32),
            pltpu.VMEM((2, IDX_PER_C, PAIR), jnp.float32),
            pltpu.VMEM((NUM_COLLISIONS, EMBED_DIM), jnp.float32),
            pltpu.VMEM((GB, EMBED_DIM), jnp.float32),
            pltpu.VMEM((EMBED_DIM, RG * GB), jnp.float32),
            pltpu.SemaphoreType.DMA,
            pltpu.SemaphoreType.DMA,
        ],
    )
    def kern(inp_hbm, wq_hbm, wr_hbm, out_hbm,
             raw_v, idxq_v, rows_v, wr_v, acc_v, out_v, sem0, sem1):
        sems = (sem0, sem1)
        wid = lax.axis_index("s") * NC + lax.axis_index("c")
        pltpu.sync_copy(wr_hbm, wr_v)
        lanes = lax.iota(jnp.int32, L)
        zf = jnp.zeros((L,), jnp.float32)
        zi = jnp.zeros((L,), jnp.int32)

        wr_vec = [[wr_v[k, pl.ds(d * L, L)] for d in range(DV)]
                  for k in range(NUM_COLLISIONS)]

        def prep(g, half, buf):
            sub = lax.rem(g, RG)

            if half == 0:
                @pl.when(sub == 0)
                def _():
                    cb = wid * BAGS_PER_W + (g // RG) * (RG * GB)
                    pltpu.sync_copy(inp_hbm.at[:, pl.ds(cb, RG * GB)], raw_v)

            coff = sub * L

            def jbody(j, st):
                n0, n1, s1, s2, s3 = st
                v = raw_v[half * CH + j, pl.ds(coff, L)]
                p = lax.shift_right_logical(v, 3)
                h = jnp.bitwise_and(lax.shift_right_logical(v, 2), 1)
                pos = jnp.where(h == 0, n0, (CH - 1) - n1)
                flat = lanes * CH + pos
                row = lax.shift_right_logical(flat * 52429, 22)
                col = flat - row * SUB
                plsc.store_scatter(idxq_v.at[buf], [row, col], p)
                n1 = n1 + h
                n0 = n0 + (1 - h)
                r = jnp.bitwise_and(v, 3)
                r2 = r * r
                s1 = s1 + r
                s2 = s2 + r2
                s3 = s3 + r2 * r
                return (n0, n1, s1, s2, s3)

            n0, _, s1, s2, s3 = lax.fori_loop(
                0, CH, jbody, (zi,) * 5, unroll=25)
            for jj in range(N_SUB):
                pltpu.async_copy(
                    wq_hbm.at[idxq_v.at[buf, jj]],
                    rows_v.at[buf, pl.ds(jj * SUB, SUB)],
                    sems[buf],
                )
            return (n0, s1, s2, s3)

        def drain(buf):
            for jj in range(N_SUB):
                pltpu.make_async_copy(
                    wq_hbm.at[idxq_v.at[buf, jj]],
                    rows_v.at[buf, pl.ds(jj * SUB, SUB)],
                    sems[buf],
                ).wait()

        def chunk_sums(buf, l, n0):
            def sbody(j, accs):
                row = l * CH + j
                off = jnp.where(j < n0, 0, EMBED_DIM)
                return tuple(
                    accs[d] + rows_v[buf, row, pl.ds(off + d * L, L)]
                    for d in range(DV)
                )
            return lax.fori_loop(0, CH, sbody, (zf,) * DV, unroll=13)

        def compute_a(stA):
            n0v = stA[0]
            drain(0)
            for l in range(GB):
                accs = chunk_sums(0, l, n0v[l])
                for d in range(DV):
                    acc_v[l, pl.ds(d * L, L)] = accs[d]

        def compute_b(g, stA, stB):
            n0v = stB[0]
            s1 = (stA[1] + stB[1]).astype(jnp.float32)
            s2 = (stA[2] + stB[2]).astype(jnp.float32)
            s3 = (stA[3] + stB[3]).astype(jnp.float32)
            c3 = (s3 - 3.0 * s2 + 2.0 * s1) * jnp.float32(1.0 / 6.0)
            c2 = (s2 - s1) * 0.5 - 3.0 * c3
            c1 = s1 - 2.0 * c2 - 3.0 * c3
            c0 = jnp.float32(HIST) - c1 - c2 - c3
            drain(1)
            inv = jnp.float32(1.0 / (HIST * HIST))
            sub = lax.rem(g, RG)
            for l in range(GB):
                cs = (c0[l], c1[l], c2[l], c3[l])
                sr = [
                    (cs[0] * wr_vec[0][d] + cs[1] * wr_vec[1][d]
                     + cs[2] * wr_vec[2][d] + cs[3] * wr_vec[3][d]) * inv
                    for d in range(DV)
                ]
                accs = chunk_sums(1, l, n0v[l])
                lv = jnp.broadcast_to(sub * L + l, (L,))
                for d in range(DV):
                    tot = (accs[d] + acc_v[l, pl.ds(d * L, L)]) * sr[d]
                    plsc.store_scatter(out_v, [lanes + d * L, lv], tot)

            @pl.when(sub == RG - 1)
            def _():
                cb = wid * BAGS_PER_W + (g // RG) * (RG * GB)
                pltpu.sync_copy(out_v, out_hbm.at[:, pl.ds(cb, RG * GB)])

        stA0 = prep(jnp.int32(0), 0, 0)

        def body(g, stA):
            stB = prep(g, 1, 1)
            compute_a(stA)
            gn = lax.select(g + 1 >= NG, jnp.int32(0), g + 1)
            stA_next = prep(gn, 0, 0)
            compute_b(g, stA, stB)
            return stA_next

        lax.fori_loop(0, NG, body, stA0)
        drain(0)

    out_t = kern(inp_t, wq_pair, weight_r)
    return out_t.T


def kernel(input, weight_q, weight_r):
    return _qr_bag(input, weight_q, weight_r)


# speedup vs baseline: 1.3223x; 1.3223x over previous
import jax
import jax.numpy as jnp
from jax import lax
from jax.experimental import pallas as pl
from jax.experimental.pallas import tpu as pltpu
from jax.experimental.pallas import tpu_sc as plsc

NUM_COLLISIONS = 4
EMBED_DIM = 64
BATCH = 16384
HIST = 50

NC, NS, L = 2, 16, 16
NW = NC * NS
BAGS_PER_W = BATCH // NW
GB = 16
NG = BAGS_PER_W // GB
CH = HIST // 2
IDX_PER_C = GB * CH
N_SUB = 5
SUB = IDX_PER_C // N_SUB
DV = EMBED_DIM // L
PAIR = 2 * EMBED_DIM
NQP = 125000
RG = 8

_mesh = plsc.VectorSubcoreMesh(core_axis_name="c", subcore_axis_name="s")


@jax.jit
def _qr_bag(inp, weight_q, weight_r):
    inp_t = inp.T
    wq_pair = weight_q.reshape(NQP, PAIR)

    @pl.kernel(
        out_type=jax.ShapeDtypeStruct((EMBED_DIM, BATCH), jnp.float32),
        mesh=_mesh,
        compiler_params=pltpu.CompilerParams(
            needs_layout_passes=False, disable_bounds_checks=True),
        scratch_types=[
            pltpu.VMEM((HIST, RG * GB), jnp.int32),
            pltpu.VMEM((2, N_SUB, SUB), jnp.int32),
            pltpu.VMEM((2, IDX_PER_C, PAIR), jnp.float32),
            pltpu.VMEM((NUM_COLLISIONS, EMBED_DIM), jnp.float32),
            pltpu.VMEM((GB, EMBED_DIM), jnp.float32),
            pltpu.VMEM((EMBED_DIM, RG * GB), jnp.float32),
            pltpu.SemaphoreType.DMA,
            pltpu.SemaphoreType.DMA,
        ],
    )
    def kern(inp_hbm, wq_hbm, wr_hbm, out_hbm,
             raw_v, idxq_v, rows_v, wr_v, acc_v, out_v, sem0, sem1):
        sems = (sem0, sem1)
        wid = lax.axis_index("s") * NC + lax.axis_index("c")
        pltpu.sync_copy(wr_hbm, wr_v)
        lanes = lax.iota(jnp.int32, L)
        zf = jnp.zeros((L,), jnp.float32)
        zi = jnp.zeros((L,), jnp.int32)

        wr_vec = [[wr_v[k, pl.ds(d * L, L)] for d in range(DV)]
                  for k in range(NUM_COLLISIONS)]

        def prep(g, half, buf):
            sub = lax.rem(g, RG)

            if half == 0:
                @pl.when(sub == 0)
                def _():
                    cb = wid * BAGS_PER_W + (g // RG) * (RG * GB)
                    pltpu.sync_copy(inp_hbm.at[:, pl.ds(cb, RG * GB)], raw_v)

            coff = sub * L

            @plsc.parallel_loop(0, CH, 1, unroll=5, carry=(zi,) * 5)
            def jloop(j, st):
                n0, n1, s1, s2, s3 = st
                v = raw_v[half * CH + j, pl.ds(coff, L)]
                p = lax.shift_right_logical(v, 3)
                h = jnp.bitwise_and(lax.shift_right_logical(v, 2), 1)
                pos = jnp.where(h == 0, n0, (CH - 1) - n1)
                flat = lanes * CH + pos
                row = lax.shift_right_logical(flat * 52429, 22)
                col = flat - row * SUB
                plsc.store_scatter(idxq_v.at[buf], [row, col], p)
                n1 = n1 + h
                n0 = n0 + (1 - h)
                r = jnp.bitwise_and(v, 3)
                r2 = r * r
                s1 = s1 + r
                s2 = s2 + r2
                s3 = s3 + r2 * r
                return (n0, n1, s1, s2, s3)

            n0, _, s1, s2, s3 = jloop
            for jj in range(N_SUB):
                pltpu.async_copy(
                    wq_hbm.at[idxq_v.at[buf, jj]],
                    rows_v.at[buf, pl.ds(jj * SUB, SUB)],
                    sems[buf],
                )
            return (n0, s1, s2, s3)

        def drain(buf):
            for jj in range(N_SUB):
                pltpu.make_async_copy(
                    wq_hbm.at[idxq_v.at[buf, jj]],
                    rows_v.at[buf, pl.ds(jj * SUB, SUB)],
                    sems[buf],
                ).wait()

        def chunk_sums(buf, l, n0):
            @plsc.parallel_loop(0, CH, 1, unroll=5, carry=(zf,) * DV)
            def sloop(j, accs):
                row = l * CH + j
                off = jnp.where(j < n0, 0, EMBED_DIM)
                return tuple(
                    accs[d] + rows_v[buf, row, pl.ds(off + d * L, L)]
                    for d in range(DV)
                )
            return sloop

        def compute_a(stA):
            n0v = stA[0]
            drain(0)
            for l in range(GB):
                accs = chunk_sums(0, l, n0v[l])
                for d in range(DV):
                    acc_v[l, pl.ds(d * L, L)] = accs[d]

        def compute_b(g, stA, stB):
            n0v = stB[0]
            s1 = (stA[1] + stB[1]).astype(jnp.float32)
            s2 = (stA[2] + stB[2]).astype(jnp.float32)
            s3 = (stA[3] + stB[3]).astype(jnp.float32)
            c3 = (s3 - 3.0 * s2 + 2.0 * s1) * jnp.float32(1.0 / 6.0)
            c2 = (s2 - s1) * 0.5 - 3.0 * c3
            c1 = s1 - 2.0 * c2 - 3.0 * c3
            c0 = jnp.float32(HIST) - c1 - c2 - c3
            drain(1)
            inv = jnp.float32(1.0 / (HIST * HIST))
            sub = lax.rem(g, RG)
            for l in range(GB):
                cs = (c0[l], c1[l], c2[l], c3[l])
                sr = [
                    (cs[0] * wr_vec[0][d] + cs[1] * wr_vec[1][d]
                     + cs[2] * wr_vec[2][d] + cs[3] * wr_vec[3][d]) * inv
                    for d in range(DV)
                ]
                accs = chunk_sums(1, l, n0v[l])
                lv = jnp.broadcast_to(sub * L + l, (L,))
                for d in range(DV):
                    tot = (accs[d] + acc_v[l, pl.ds(d * L, L)]) * sr[d]
                    plsc.store_scatter(out_v, [lanes + d * L, lv], tot)

            @pl.when(sub == RG - 1)
            def _():
                cb = wid * BAGS_PER_W + (g // RG) * (RG * GB)
                pltpu.sync_copy(out_v, out_hbm.at[:, pl.ds(cb, RG * GB)])

        stA0 = prep(jnp.int32(0), 0, 0)

        def body(g, stA):
            stB = prep(g, 1, 1)
            compute_a(stA)
            gn = lax.select(g + 1 >= NG, jnp.int32(0), g + 1)
            stA_next = prep(gn, 0, 0)
            compute_b(g, stA, stB)
            return stA_next

        lax.fori_loop(0, NG, body, stA0)
        drain(0)

    out_t = kern(inp_t, wq_pair, weight_r)
    return out_t.T


def kernel(input, weight_q, weight_r):
    return _qr_bag(input, weight_q, weight_r)


# speedup vs baseline: 1.3781x; 1.0422x over previous
import jax
import jax.numpy as jnp
from jax import lax
from jax.experimental import pallas as pl
from jax.experimental.pallas import tpu as pltpu
from jax.experimental.pallas import tpu_sc as plsc

NUM_COLLISIONS = 4
EMBED_DIM = 64
BATCH = 16384
HIST = 50

NC, NS, L = 2, 16, 16
NW = NC * NS
BAGS_PER_W = BATCH // NW
GB = 16
NG = BAGS_PER_W // GB
CH = HIST // 2
IDX_PER_C = GB * CH
N_SUB = 5
SUB = IDX_PER_C // N_SUB
JPR = SUB // L
DV = EMBED_DIM // L
WIDE = 2 * EMBED_DIM
RG = 8

_mesh = plsc.VectorSubcoreMesh(core_axis_name="c", subcore_axis_name="s")


@jax.jit
def _qr_bag(inp, weight_q, weight_r):
    inp_t = inp.T
    wq_wide = jnp.pad(weight_q, ((0, 0), (0, EMBED_DIM)))

    @pl.kernel(
        out_type=jax.ShapeDtypeStruct((EMBED_DIM, BATCH), jnp.float32),
        mesh=_mesh,
        compiler_params=pltpu.CompilerParams(
            needs_layout_passes=False, disable_bounds_checks=True),
        scratch_types=[
            pltpu.VMEM((HIST, RG * GB), jnp.int32),
            pltpu.VMEM((2, N_SUB, SUB), jnp.int32),
            pltpu.VMEM((2, IDX_PER_C, WIDE), jnp.float32),
            pltpu.VMEM((NUM_COLLISIONS, EMBED_DIM), jnp.float32),
            pltpu.VMEM((GB, EMBED_DIM), jnp.float32),
            pltpu.VMEM((EMBED_DIM, RG * GB), jnp.float32),
            pltpu.SemaphoreType.DMA,
            pltpu.SemaphoreType.DMA,
        ],
    )
    def kern(inp_hbm, wq_hbm, wr_hbm, out_hbm,
             raw_v, idxq_v, rows_v, wr_v, acc_v, out_v, sem0, sem1):
        sems = (sem0, sem1)
        wid = lax.axis_index("s") * NC + lax.axis_index("c")
        pltpu.sync_copy(wr_hbm, wr_v)
        lanes = lax.iota(jnp.int32, L)
        zf = jnp.zeros((L,), jnp.float32)
        zi = jnp.zeros((L,), jnp.int32)

        wr_vec = [[wr_v[k, pl.ds(d * L, L)] for d in range(DV)]
                  for k in range(NUM_COLLISIONS)]

        def prep(g, half, buf):
            sub = lax.rem(g, RG)

            if half == 0:
                @pl.when(sub == 0)
                def _():
                    cb = wid * BAGS_PER_W + (g // RG) * (RG * GB)
                    pltpu.sync_copy(inp_hbm.at[:, pl.ds(cb, RG * GB)], raw_v)

            coff = sub * L

            @plsc.parallel_loop(0, CH, 1, unroll=5, carry=(zi,) * 3)
            def jloop(j, st):
                s1, s2, s3 = st
                v = raw_v[half * CH + j, pl.ds(coff, L)]
                q = lax.shift_right_logical(v, 2)
                jr = j // JPR
                jc = (j - jr * JPR) * L
                plsc.store_scatter(
                    idxq_v.at[buf],
                    [jnp.broadcast_to(jr, (L,)), jc + lanes], q)
                r = jnp.bitwise_and(v, 3)
                r2 = r * r
                return (s1 + r, s2 + r2, s3 + r2 * r)

            for jj in range(N_SUB):
                pltpu.async_copy(
                    wq_hbm.at[idxq_v.at[buf, jj]],
                    rows_v.at[buf, pl.ds(jj * SUB, SUB)],
                    sems[buf],
                )
            return jloop

        def drain(buf):
            for jj in range(N_SUB):
                pltpu.make_async_copy(
                    wq_hbm.at[idxq_v.at[buf, jj]],
                    rows_v.at[buf, pl.ds(jj * SUB, SUB)],
                    sems[buf],
                ).wait()

        def chunk_sums(buf, l):
            @plsc.parallel_loop(0, CH, 1, unroll=5, carry=(zf,) * DV)
            def sloop(j, accs):
                row = j * L + l
                return tuple(
                    accs[d] + rows_v[buf, row, pl.ds(d * L, L)]
                    for d in range(DV)
                )
            return sloop

        def compute_a():
            drain(0)
            for l in range(GB):
                accs = chunk_sums(0, l)
                for d in range(DV):
                    acc_v[l, pl.ds(d * L, L)] = accs[d]

        def compute_b(g, stA, stB):
            s1 = (stA[0] + stB[0]).astype(jnp.float32)
            s2 = (stA[1] + stB[1]).astype(jnp.float32)
            s3 = (stA[2] + stB[2]).astype(jnp.float32)
            c3 = (s3 - 3.0 * s2 + 2.0 * s1) * jnp.float32(1.0 / 6.0)
            c2 = (s2 - s1) * 0.5 - 3.0 * c3
            c1 = s1 - 2.0 * c2 - 3.0 * c3
            c0 = jnp.float32(HIST) - c1 - c2 - c3
            drain(1)
            inv = jnp.float32(1.0 / (HIST * HIST))
            sub = lax.rem(g, RG)
            for l in range(GB):
                cs = (c0[l], c1[l], c2[l], c3[l])
                sr = [
                    (cs[0] * wr_vec[0][d] + cs[1] * wr_vec[1][d]
                     + cs[2] * wr_vec[2][d] + cs[3] * wr_vec[3][d]) * inv
                    for d in range(DV)
                ]
                accs = chunk_sums(1, l)
                lv = jnp.broadcast_to(sub * L + l, (L,))
                for d in range(DV):
                    tot = (accs[d] + acc_v[l, pl.ds(d * L, L)]) * sr[d]
                    plsc.store_scatter(out_v, [lanes + d * L, lv], tot)

            @pl.when(sub == RG - 1)
            def _():
                cb = wid * BAGS_PER_W + (g // RG) * (RG * GB)
                pltpu.sync_copy(out_v, out_hbm.at[:, pl.ds(cb, RG * GB)])

        stA0 = prep(jnp.int32(0), 0, 0)

        def body(g, stA):
            stB = prep(g, 1, 1)
            compute_a()
            gn = lax.select(g + 1 >= NG, jnp.int32(0), g + 1)
            stA_next = prep(gn, 0, 0)
            compute_b(g, stA, stB)
            return stA_next

        lax.fori_loop(0, NG, body, stA0)
        drain(0)

    out_t = kern(inp_t, wq_wide, weight_r)
    return out_t.T


def kernel(input, weight_q, weight_r):
    return _qr_bag(input, weight_q, weight_r)


# speedup vs baseline: 1.3826x; 1.0033x over previous
import jax
import jax.numpy as jnp
from jax import lax
from jax.experimental import pallas as pl
from jax.experimental.pallas import tpu as pltpu
from jax.experimental.pallas import tpu_sc as plsc

NUM_COLLISIONS = 4
EMBED_DIM = 64
BATCH = 16384
HIST = 50

NC, NS, L = 2, 16, 16
NW = NC * NS
BAGS_PER_W = BATCH // NW
GB = 16
NG = BAGS_PER_W // GB
CH = HIST // 2
IDX_PER_C = GB * CH
N_SUB = 5
SUB = IDX_PER_C // N_SUB
JPR = SUB // L
DV = EMBED_DIM // L
WIDE = 2 * EMBED_DIM
RG = 8

_mesh = plsc.VectorSubcoreMesh(core_axis_name="c", subcore_axis_name="s")


@jax.jit
def _qr_bag(inp, weight_q, weight_r):
    inp_t = inp.T
    wq_wide = jnp.pad(weight_q, ((0, 0), (0, EMBED_DIM)))

    @pl.kernel(
        out_type=jax.ShapeDtypeStruct((EMBED_DIM, BATCH), jnp.float32),
        mesh=_mesh,
        compiler_params=pltpu.CompilerParams(
            needs_layout_passes=False, disable_bounds_checks=True),
        scratch_types=[
            pltpu.VMEM((HIST, RG * GB), jnp.int32),
            pltpu.VMEM((2, N_SUB, SUB), jnp.int32),
            pltpu.VMEM((2, IDX_PER_C, WIDE), jnp.float32),
            pltpu.VMEM((NUM_COLLISIONS, EMBED_DIM), jnp.float32),
            pltpu.VMEM((GB, EMBED_DIM), jnp.float32),
            pltpu.VMEM((EMBED_DIM, RG * GB), jnp.float32),
            pltpu.SemaphoreType.DMA,
            pltpu.SemaphoreType.DMA,
        ],
    )
    def kern(inp_hbm, wq_hbm, wr_hbm, out_hbm,
             raw_v, idxq_v, rows_v, wr_v, acc_v, out_v, sem0, sem1):
        sems = (sem0, sem1)
        wid = lax.axis_index("s") * NC + lax.axis_index("c")
        pltpu.sync_copy(wr_hbm, wr_v)
        lanes = lax.iota(jnp.int32, L)
        zf = jnp.zeros((L,), jnp.float32)
        zi = jnp.zeros((L,), jnp.int32)

        wr_vec = [[wr_v[k, pl.ds(d * L, L)] for d in range(DV)]
                  for k in range(NUM_COLLISIONS)]

        def prep(g, half, buf):
            sub = lax.rem(g, RG)

            if half == 0:
                @pl.when(sub == 0)
                def _():
                    cb = wid * BAGS_PER_W + (g // RG) * (RG * GB)
                    pltpu.sync_copy(inp_hbm.at[:, pl.ds(cb, RG * GB)], raw_v)

            coff = sub * L

            @plsc.parallel_loop(0, CH, 1, unroll=5, carry=(zi,) * 3)
            def jloop(j, st):
                s1, s2, s3 = st
                v = raw_v[half * CH + j, pl.ds(coff, L)]
                q = lax.shift_right_logical(v, 2)
                jr = j // JPR
                jc = (j - jr * JPR) * L
                plsc.store_scatter(
                    idxq_v.at[buf],
                    [jnp.broadcast_to(jr, (L,)), jc + lanes], q)
                r = jnp.bitwise_and(v, 3)
                r2 = r * r
                return (s1 + r, s2 + r2, s3 + r2 * r)

            for jj in range(N_SUB):
                pltpu.async_copy(
                    wq_hbm.at[idxq_v.at[buf, jj]],
                    rows_v.at[buf, pl.ds(jj * SUB, SUB)],
                    sems[buf],
                )
            return jloop

        def drain(buf):
            for jj in range(N_SUB):
                pltpu.make_async_copy(
                    wq_hbm.at[idxq_v.at[buf, jj]],
                    rows_v.at[buf, pl.ds(jj * SUB, SUB)],
                    sems[buf],
                ).wait()

        def chunk_sums(buf, l):
            @plsc.parallel_loop(0, CH - 1, 4, carry=(zf,) * (4 * DV))
            def sloop(j, accs):
                base = j * L + l
                return tuple(
                    accs[t * DV + d]
                    + rows_v[buf, base + t * L, pl.ds(d * L, L)]
                    for t in range(4) for d in range(DV)
                )
            last = (CH - 1) * L + l
            return tuple(
                sloop[d] + sloop[DV + d] + sloop[2 * DV + d]
                + sloop[3 * DV + d] + rows_v[buf, last, pl.ds(d * L, L)]
                for d in range(DV)
            )

        def compute_a():
            drain(0)
            for l in range(GB):
                accs = chunk_sums(0, l)
                for d in range(DV):
                    acc_v[l, pl.ds(d * L, L)] = accs[d]

        def compute_b(g, stA, stB):
            s1 = (stA[0] + stB[0]).astype(jnp.float32)
            s2 = (stA[1] + stB[1]).astype(jnp.float32)
            s3 = (stA[2] + stB[2]).astype(jnp.float32)
            c3 = (s3 - 3.0 * s2 + 2.0 * s1) * jnp.float32(1.0 / 6.0)
            c2 = (s2 - s1) * 0.5 - 3.0 * c3
            c1 = s1 - 2.0 * c2 - 3.0 * c3
            c0 = jnp.float32(HIST) - c1 - c2 - c3
            drain(1)
            inv = jnp.float32(1.0 / (HIST * HIST))
            sub = lax.rem(g, RG)
            for l in range(GB):
                cs = (c0[l], c1[l], c2[l], c3[l])
                sr = [
                    (cs[0] * wr_vec[0][d] + cs[1] * wr_vec[1][d]
                     + cs[2] * wr_vec[2][d] + cs[3] * wr_vec[3][d]) * inv
                    for d in range(DV)
                ]
                accs = chunk_sums(1, l)
                lv = jnp.broadcast_to(sub * L + l, (L,))
                for d in range(DV):
                    tot = (accs[d] + acc_v[l, pl.ds(d * L, L)]) * sr[d]
                    plsc.store_scatter(out_v, [lanes + d * L, lv], tot)

            @pl.when(sub == RG - 1)
            def _():
                cb = wid * BAGS_PER_W + (g // RG) * (RG * GB)
                pltpu.sync_copy(out_v, out_hbm.at[:, pl.ds(cb, RG * GB)])

        stA0 = prep(jnp.int32(0), 0, 0)

        def body(g, stA):
            stB = prep(g, 1, 1)
            compute_a()
            gn = lax.select(g + 1 >= NG, jnp.int32(0), g + 1)
            stA_next = prep(gn, 0, 0)
            compute_b(g, stA, stB)
            return stA_next

        lax.fori_loop(0, NG, body, stA0)
        drain(0)

    out_t = kern(inp_t, wq_wide, weight_r)
    return out_t.T


def kernel(input, weight_q, weight_r):
    return _qr_bag(input, weight_q, weight_r)


# speedup vs baseline: 1.6021x; 1.1587x over previous
import jax
import jax.numpy as jnp
from jax import lax
from jax.experimental import pallas as pl
from jax.experimental.pallas import tpu as pltpu
from jax.experimental.pallas import tpu_sc as plsc

NUM_COLLISIONS = 4
EMBED_DIM = 64
BATCH = 16384
HIST = 50

NC, NS, L = 2, 16, 16
NW = NC * NS
BAGS_PER_W = BATCH // NW
GB = 16
NG = BAGS_PER_W // GB
IDX_PER_G = GB * HIST
N_SUB = 10
SUB = IDX_PER_G // N_SUB
JPR = SUB // L
DV = EMBED_DIM // L

_mesh = plsc.VectorSubcoreMesh(core_axis_name="c", subcore_axis_name="s")


@jax.jit
def _qr_bag(inp, weight_q, weight_r):
    inp_flat = jnp.concatenate(
        [inp.reshape(-1), jnp.zeros((IDX_PER_G,), jnp.int32)])

    @pl.kernel(
        out_type=jax.ShapeDtypeStruct((BATCH, EMBED_DIM), jnp.float32),
        mesh=_mesh,
        compiler_params=pltpu.CompilerParams(
            needs_layout_passes=False, use_tc_tiling_on_sc=False,
            disable_bounds_checks=True),
        scratch_types=[
            pltpu.VMEM((IDX_PER_G,), jnp.int32),
            pltpu.VMEM((2, N_SUB, SUB), jnp.int32),
            pltpu.VMEM((2, IDX_PER_G, EMBED_DIM), jnp.float32),
            pltpu.VMEM((NUM_COLLISIONS, EMBED_DIM), jnp.float32),
            pltpu.VMEM((GB, EMBED_DIM), jnp.float32),
            pltpu.SemaphoreType.DMA,
            pltpu.SemaphoreType.DMA,
        ],
    )
    def kern(inp_hbm, wq_hbm, wr_hbm, out_hbm,
             raw_v, idxq_v, rows_v, wr_v, out_v, sem0, sem1):
        sems = (sem0, sem1)
        wid = lax.axis_index("s") * NC + lax.axis_index("c")
        pltpu.sync_copy(wr_hbm, wr_v)
        lanes = lax.iota(jnp.int32, L)
        zf = jnp.zeros((L,), jnp.float32)
        zi = jnp.zeros((L,), jnp.int32)

        wr_vec = [[wr_v[k, pl.ds(d * L, L)] for d in range(DV)]
                  for k in range(NUM_COLLISIONS)]

        def prep(g, buf):
            base = wid * BAGS_PER_W + g * GB
            pltpu.sync_copy(inp_hbm.at[pl.ds(base * HIST, IDX_PER_G)], raw_v)

            @plsc.parallel_loop(0, HIST, 1, unroll=5, carry=(zi,) * 3)
            def jloop(j, st):
                s1, s2, s3 = st
                v = plsc.load_gather(raw_v, [lanes * HIST + j])
                q = lax.shift_right_logical(v, 2)
                idxq_v[buf, j // JPR, pl.ds((j % JPR) * L, L)] = q
                r = jnp.bitwise_and(v, 3)
                r2 = r * r
                return (s1 + r, s2 + r2, s3 + r2 * r)

            for jj in range(N_SUB):
                pltpu.async_copy(
                    wq_hbm.at[idxq_v.at[buf, jj]],
                    rows_v.at[buf, pl.ds(jj * SUB, SUB)],
                    sems[buf],
                )
            return jloop

        def drain(buf):
            for jj in range(N_SUB):
                pltpu.make_async_copy(
                    wq_hbm.at[idxq_v.at[buf, jj]],
                    rows_v.at[buf, pl.ds(jj * SUB, SUB)],
                    sems[buf],
                ).wait()

        def bag_sums(buf, l):
            @plsc.parallel_loop(0, HIST - 2, 4, carry=(zf,) * (4 * DV))
            def sloop(j, accs):
                base = j * L + l
                return tuple(
                    accs[t * DV + d]
                    + rows_v[buf, base + t * L, pl.ds(d * L, L)]
                    for t in range(4) for d in range(DV)
                )
            r48 = (HIST - 2) * L + l
            r49 = (HIST - 1) * L + l
            return tuple(
                sloop[d] + sloop[DV + d] + sloop[2 * DV + d]
                + sloop[3 * DV + d] + rows_v[buf, r48, pl.ds(d * L, L)]
                + rows_v[buf, r49, pl.ds(d * L, L)]
                for d in range(DV)
            )

        def compute(g, buf, st):
            base = wid * BAGS_PER_W + g * GB
            s1 = st[0].astype(jnp.float32)
            s2 = st[1].astype(jnp.float32)
            s3 = st[2].astype(jnp.float32)
            c3 = (s3 - 3.0 * s2 + 2.0 * s1) * jnp.float32(1.0 / 6.0)
            c2 = (s2 - s1) * 0.5 - 3.0 * c3
            c1 = s1 - 2.0 * c2 - 3.0 * c3
            c0 = jnp.float32(HIST) - c1 - c2 - c3
            drain(buf)
            inv = jnp.float32(1.0 / (HIST * HIST))
            for l in range(GB):
                cs = (c0[l], c1[l], c2[l], c3[l])
                sr = [
                    (cs[0] * wr_vec[0][d] + cs[1] * wr_vec[1][d]
                     + cs[2] * wr_vec[2][d] + cs[3] * wr_vec[3][d]) * inv
                    for d in range(DV)
                ]
                accs = bag_sums(buf, l)
                for d in range(DV):
                    out_v[l, pl.ds(d * L, L)] = accs[d] * sr[d]

            pltpu.sync_copy(out_v, out_hbm.at[pl.ds(base, GB)])

        st0 = prep(jnp.int32(0), 0)

        def body(gg, st_cur):
            g0 = gg * 2
            st_n1 = prep(g0 + 1, 1)
            compute(g0, 0, st_cur)
            st_n2 = prep(g0 + 2, 0)
            compute(g0 + 1, 1, st_n1)
            return st_n2

        lax.fori_loop(0, NG // 2, body, st0)
        drain(0)

    return kern(inp_flat, weight_q, weight_r)


def kernel(input, weight_q, weight_r):
    return _qr_bag(input, weight_q, weight_r)
